# baseline (speedup 1.0000x reference)
import dataclasses
import functools

import jax
import jax.numpy as jnp
from jax import lax
from jax.experimental import pallas as pl
from jax.experimental.pallas import tpu as pltpu
from jax.experimental.pallas import tpu_sc as plsc

_B = 16384
_D = 64
_W = 128
_NC = 2
_NS = 16
_NW = _NC * _NS
_BPW = _B // _NW
_ROWCHUNK = 128
_NCHUNK = _BPW // _ROWCHUNK
_LANES = 16


def _make_sc_dot():
    mesh = plsc.VectorSubcoreMesh(core_axis_name="c", subcore_axis_name="s")
    cp = pltpu.CompilerParams()
    if "needs_layout_passes" in pltpu.CompilerParams.__dataclass_fields__:
        cp = dataclasses.replace(cp, needs_layout_passes=False)
    if "use_tc_tiling_on_sc" in pltpu.CompilerParams.__dataclass_fields__:
        cp = dataclasses.replace(cp, use_tc_tiling_on_sc=True)

    @functools.partial(
        pl.kernel,
        mesh=mesh,
        compiler_params=cp,
        out_type=jax.ShapeDtypeStruct((_B,), jnp.float32),
        scratch_types=[
            pltpu.VMEM((_ROWCHUNK, _W), jnp.float32),
            pltpu.VMEM((_ROWCHUNK, _W), jnp.float32),
            pltpu.VMEM((_ROWCHUNK, _W), jnp.float32),
            pltpu.VMEM((_ROWCHUNK, _W), jnp.float32),
            pltpu.VMEM((_BPW,), jnp.float32),
            pltpu.VMEM((2 * _BPW,), jnp.int32),
            pltpu.VMEM((_ROWCHUNK * _W,), jnp.float32),
            pltpu.SemaphoreType.DMA,
            pltpu.SemaphoreType.DMA,
        ],
    )
    def sc_dot(uid_hbm, iid_hbm, ut_hbm, it_hbm, out_hbm,
               ub0, ub1, ib0, ib1, out_v, idx_v, dummy_v,
               sem0, sem1):
        ubufs = (ub0, ub1)
        ibufs = (ib0, ib1)
        sems = (sem0, sem1)
        wid = lax.axis_index("s") * _NC + lax.axis_index("c")
        base = wid * _BPW

        pltpu.sync_copy(uid_hbm.at[pl.ds(base, _BPW)], idx_v.at[pl.ds(0, _BPW)])
        pltpu.sync_copy(iid_hbm.at[pl.ds(base, _BPW)], idx_v.at[pl.ds(_BPW, _BPW)])

        def fire(c):
            ub, ib, sem = ubufs[c % 2], ibufs[c % 2], sems[c % 2]

            @pl.loop(0, _ROWCHUNK, step=_LANES)
            def _(j0):
                i0 = c * _ROWCHUNK + j0
                uvec = idx_v[pl.ds(i0, _LANES)]
                ivec = idx_v[pl.ds(_BPW + i0, _LANES)]
                for l in range(_LANES):
                    dst = pl.ds(j0 + l, 1)
                    pltpu.async_copy(
                        ut_hbm.at[uvec[l] // 2], ub.at[j0 + l], sem)
                    pltpu.async_copy(
                        it_hbm.at[ivec[l] // 2], ib.at[j0 + l], sem)

        def drain(c):
            sem = sems[c % 2]
            dummy_src = out_hbm.at[pl.ds(0, _ROWCHUNK * _W)]
            pltpu.make_async_copy(dummy_src, dummy_v, sem).wait()
            pltpu.make_async_copy(dummy_src, dummy_v, sem).wait()

        def compute(c):
            ub, ib = ubufs[c % 2], ibufs[c % 2]

            @pl.loop(0, _ROWCHUNK, step=_LANES)
            def _(r0):
                i0 = c * _ROWCHUNK + r0
                uvec = idx_v[pl.ds(i0, _LANES)]
                ivec = idx_v[pl.ds(_BPW + i0, _LANES)]
                uhalf = (uvec & 1) * _D
                ihalf = (ivec & 1) * _D
                rows = r0 + lax.iota(jnp.int32, _LANES)
                acc = None
                for d in range(_D):
                    u = plsc.load_gather(ub, [rows, uhalf + d])
                    v = plsc.load_gather(ib, [rows, ihalf + d])
                    prod = u * v
                    acc = prod if acc is None else acc + prod
                out_v[pl.ds(i0, _LANES)] = acc

        fire(0)
        for c in range(_NCHUNK):
            if c + 1 < _NCHUNK:
                fire(c + 1)
            drain(c)
            compute(c)

        pltpu.sync_copy(out_v, out_hbm.at[pl.ds(base, _BPW)])

    return sc_dot


_sc_dot = _make_sc_dot()


def kernel(input_user_id, input_item_id, user_table, item_table):
    uid = input_user_id.reshape(_B).astype(jnp.int32)
    iid = input_item_id.reshape(_B).astype(jnp.int32)
    ut2 = user_table.reshape(-1, _W)
    it2 = item_table.reshape(-1, _W)
    out = _sc_dot(uid, iid, ut2, it2)
    return out.reshape(_B, 1)


# speedup vs baseline: 2.5099x; 2.5099x over previous
import dataclasses
import functools

import jax
import jax.numpy as jnp
from jax import lax
from jax.experimental import pallas as pl
from jax.experimental.pallas import tpu as pltpu
from jax.experimental.pallas import tpu_sc as plsc

_B = 16384
_D = 64
_W = 128
_NC = 2
_NS = 16
_NW = _NC * _NS
_BPW = _B // _NW
_LANES = 16
_G = _BPW // _LANES
_NBUF = 4


def _make_sc_dot():
    mesh = plsc.VectorSubcoreMesh(core_axis_name="c", subcore_axis_name="s")
    cp = pltpu.CompilerParams()
    if "needs_layout_passes" in pltpu.CompilerParams.__dataclass_fields__:
        cp = dataclasses.replace(cp, needs_layout_passes=False)
    if "use_tc_tiling_on_sc" in pltpu.CompilerParams.__dataclass_fields__:
        cp = dataclasses.replace(cp, use_tc_tiling_on_sc=True)

    iota16 = lambda: lax.iota(jnp.int32, _LANES)

    @functools.partial(
        pl.kernel,
        mesh=mesh,
        compiler_params=cp,
        out_type=jax.ShapeDtypeStruct((_B,), jnp.float32),
        scratch_types=[
            pltpu.VMEM((_D, _W), jnp.float32),
            pltpu.VMEM((_D, _W), jnp.float32),
            pltpu.VMEM((_D, _W), jnp.float32),
            pltpu.VMEM((_D, _W), jnp.float32),
            pltpu.VMEM((8, _W), jnp.float32),
            pltpu.VMEM((_BPW,), jnp.float32),
            pltpu.VMEM((2 * _BPW,), jnp.int32),
            pltpu.VMEM((_D * _W,), jnp.float32),
            pltpu.SemaphoreType.DMA,
            pltpu.SemaphoreType.DMA,
            pltpu.SemaphoreType.DMA,
            pltpu.SemaphoreType.DMA,
        ],
    )
    def sc_dot(uid_hbm, iid_hbm, utT_hbm, it_hbm, out_hbm,
               ublk0, ublk1, ublk2, ublk3, irows, out_v, idx_v, dummy_v,
               sem0, sem1, sem2, sem3):
        ublks = (ublk0, ublk1, ublk2, ublk3)
        sems = (sem0, sem1, sem2, sem3)
        wid = lax.axis_index("s") * _NC + lax.axis_index("c")
        base = wid * _BPW

        pltpu.sync_copy(uid_hbm.at[pl.ds(base, _BPW)], idx_v.at[pl.ds(0, _BPW)])
        pltpu.sync_copy(iid_hbm.at[pl.ds(base, _BPW)], idx_v.at[pl.ds(_BPW, _BPW)])

        def fire(slot, uidx, iidx):
            sem = sems[slot]
            blkoff = pl.multiple_of((uidx >> 7) * _W, _W)
            for a in range(_D // 8):
                pltpu.async_copy(
                    utT_hbm.at[pl.ds(8 * a, 8), pl.ds(blkoff, _W)],
                    ublks[slot].at[pl.ds(8 * a, 8), :], sem)
            pltpu.async_copy(it_hbm.at[iidx], irows.at[slot, pl.ds(0, _D)], sem)

        def drain(slot):
            sem = sems[slot]
            pltpu.make_async_copy(
                out_hbm.at[pl.ds(0, _D * _W)], dummy_v, sem).wait()
            pltpu.make_async_copy(
                out_hbm.at[pl.ds(0, _D)], dummy_v.at[pl.ds(0, _D)], sem).wait()

        def extract_dot(slot, e_scalar):
            e16 = jnp.full((_LANES,), e_scalar & (_W - 1), jnp.int32)
            s16 = jnp.full((_LANES,), slot, jnp.int32)
            acc = None
            for c in range(_D // _LANES):
                feats = c * _LANES + iota16()
                u = plsc.load_gather(ublks[slot], [feats, e16])
                v = plsc.load_gather(irows, [s16, feats])
                prod = u * v
                acc = prod if acc is None else acc + prod
            return jnp.sum(acc)

        def group_body(g, last):
            uvec = idx_v[pl.ds(g * _LANES, _LANES)]
            ivec = idx_v[pl.ds(_BPW + g * _LANES, _LANES)]
            if not last:
                uvn = idx_v[pl.ds((g + 1) * _LANES, _LANES)]
                ivn = idx_v[pl.ds(_BPW + (g + 1) * _LANES, _LANES)]
            vecacc = jnp.zeros((_LANES,), jnp.float32)
            for l in range(_LANES):
                slot = l % _NBUF
                drain(slot)
                s = extract_dot(slot, uvec[l])
                vecacc = jnp.where(iota16() == l, s, vecacc)
                if last:
                    if l < _LANES - _NBUF:
                        fire(slot, uvec[l + _NBUF], ivec[l + _NBUF])
                elif l < _LANES - _NBUF:
                    fire(slot, uvec[l + _NBUF], ivec[l + _NBUF])
                else:
                    fire(slot, uvn[l - (_LANES - _NBUF)],
                         ivn[l - (_LANES - _NBUF)])
            out_v[pl.ds(g * _LANES, _LANES)] = vecacc

        uvec0 = idx_v[pl.ds(0, _LANES)]
        ivec0 = idx_v[pl.ds(_BPW, _LANES)]
        for l in range(_NBUF):
            fire(l, uvec0[l], ivec0[l])

        @pl.loop(0, _G - 1)
        def _(g):
            group_body(g, last=False)

        group_body(_G - 1, last=True)

        pltpu.sync_copy(out_v, out_hbm.at[pl.ds(base, _BPW)])

    return sc_dot


_sc_dot = _make_sc_dot()


def kernel(input_user_id, input_item_id, user_table, item_table):
    uid = input_user_id.reshape(_B).astype(jnp.int32)
    iid = input_item_id.reshape(_B).astype(jnp.int32)
    out = _sc_dot(uid, iid, user_table.T, item_table)
    return out.reshape(_B, 1)


# speedup vs baseline: 2.8881x; 1.1507x over previous
import dataclasses
import functools

import jax
import jax.numpy as jnp
from jax import lax
from jax.experimental import pallas as pl
from jax.experimental.pallas import tpu as pltpu
from jax.experimental.pallas import tpu_sc as plsc

_B = 16384
_D = 64
_W = 128
_NC = 2
_NS = 16
_NW = _NC * _NS
_BPW = _B // _NW
_LANES = 16
_G = _BPW // _LANES
_NBUF = 8


def _make_sc_dot():
    mesh = plsc.VectorSubcoreMesh(core_axis_name="c", subcore_axis_name="s")
    cp = pltpu.CompilerParams()
    if "needs_layout_passes" in pltpu.CompilerParams.__dataclass_fields__:
        cp = dataclasses.replace(cp, needs_layout_passes=False)
    if "use_tc_tiling_on_sc" in pltpu.CompilerParams.__dataclass_fields__:
        cp = dataclasses.replace(cp, use_tc_tiling_on_sc=True)

    iota16 = lambda: lax.iota(jnp.int32, _LANES)

    @functools.partial(
        pl.kernel,
        mesh=mesh,
        compiler_params=cp,
        out_type=jax.ShapeDtypeStruct((_B,), jnp.float32),
        scratch_types=[
            pltpu.VMEM((_D, _W), jnp.float32),
            pltpu.VMEM((_D, _W), jnp.float32),
            pltpu.VMEM((_D, _W), jnp.float32),
            pltpu.VMEM((_D, _W), jnp.float32),
            pltpu.VMEM((_D, _W), jnp.float32),
            pltpu.VMEM((_D, _W), jnp.float32),
            pltpu.VMEM((_D, _W), jnp.float32),
            pltpu.VMEM((_D, _W), jnp.float32),
            pltpu.VMEM((8, _W), jnp.float32),
            pltpu.VMEM((_BPW,), jnp.float32),
            pltpu.VMEM((2 * _BPW,), jnp.int32),
            pltpu.VMEM((_D * _W,), jnp.float32),
            pltpu.SemaphoreType.DMA,
            pltpu.SemaphoreType.DMA,
            pltpu.SemaphoreType.DMA,
            pltpu.SemaphoreType.DMA,
            pltpu.SemaphoreType.DMA,
            pltpu.SemaphoreType.DMA,
            pltpu.SemaphoreType.DMA,
            pltpu.SemaphoreType.DMA,
        ],
    )
    def sc_dot(uid_hbm, iid_hbm, utT_hbm, it_hbm, out_hbm,
               ublk0, ublk1, ublk2, ublk3, ublk4, ublk5, ublk6, ublk7,
               irows, out_v, idx_v, dummy_v,
               sem0, sem1, sem2, sem3, sem4, sem5, sem6, sem7):
        ublks = (ublk0, ublk1, ublk2, ublk3, ublk4, ublk5, ublk6, ublk7)
        sems = (sem0, sem1, sem2, sem3, sem4, sem5, sem6, sem7)
        wid = lax.axis_index("s") * _NC + lax.axis_index("c")
        base = wid * _BPW

        pltpu.sync_copy(uid_hbm.at[pl.ds(base, _BPW)], idx_v.at[pl.ds(0, _BPW)])
        pltpu.sync_copy(iid_hbm.at[pl.ds(base, _BPW)], idx_v.at[pl.ds(_BPW, _BPW)])

        def fire(slot, uidx, iidx):
            sem = sems[slot]
            blkoff = pl.multiple_of((uidx >> 7) * _W, _W)
            for a in range(_D // 8):
                pltpu.async_copy(
                    utT_hbm.at[pl.ds(8 * a, 8), pl.ds(blkoff, _W)],
                    ublks[slot].at[pl.ds(8 * a, 8), :], sem)
            pltpu.async_copy(it_hbm.at[iidx], irows.at[slot, pl.ds(0, _D)], sem)

        def drain(slot):
            sem = sems[slot]
            pltpu.make_async_copy(
                out_hbm.at[pl.ds(0, _D * _W)], dummy_v, sem).wait()
            pltpu.make_async_copy(
                out_hbm.at[pl.ds(0, _D)], dummy_v.at[pl.ds(0, _D)], sem).wait()

        def extract_dot(slot, e_scalar):
            e16 = jnp.full((_LANES,), e_scalar & (_W - 1), jnp.int32)
            s16 = jnp.full((_LANES,), slot, jnp.int32)
            acc = None
            for c in range(_D // _LANES):
                feats = c * _LANES + iota16()
                u = plsc.load_gather(ublks[slot], [feats, e16])
                v = plsc.load_gather(irows, [s16, feats])
                prod = u * v
                acc = prod if acc is None else acc + prod
            return jnp.sum(acc)

        def group_body(g, last):
            uvec = idx_v[pl.ds(g * _LANES, _LANES)]
            ivec = idx_v[pl.ds(_BPW + g * _LANES, _LANES)]
            if not last:
                uvn = idx_v[pl.ds((g + 1) * _LANES, _LANES)]
                ivn = idx_v[pl.ds(_BPW + (g + 1) * _LANES, _LANES)]
            vecacc = jnp.zeros((_LANES,), jnp.float32)
            for l in range(_LANES):
                slot = l % _NBUF
                drain(slot)
                s = extract_dot(slot, uvec[l])
                vecacc = jnp.where(iota16() == l, s, vecacc)
                if last:
                    if l < _LANES - _NBUF:
                        fire(slot, uvec[l + _NBUF], ivec[l + _NBUF])
                elif l < _LANES - _NBUF:
                    fire(slot, uvec[l + _NBUF], ivec[l + _NBUF])
                else:
                    fire(slot, uvn[l - (_LANES - _NBUF)],
                         ivn[l - (_LANES - _NBUF)])
            out_v[pl.ds(g * _LANES, _LANES)] = vecacc

        uvec0 = idx_v[pl.ds(0, _LANES)]
        ivec0 = idx_v[pl.ds(_BPW, _LANES)]
        for l in range(_NBUF):
            fire(l, uvec0[l], ivec0[l])

        @pl.loop(0, _G - 1)
        def _(g):
            group_body(g, last=False)

        group_body(_G - 1, last=True)

        pltpu.sync_copy(out_v, out_hbm.at[pl.ds(base, _BPW)])

    return sc_dot


_sc_dot = _make_sc_dot()


def kernel(input_user_id, input_item_id, user_table, item_table):
    uid = input_user_id.reshape(_B).astype(jnp.int32)
    iid = input_item_id.reshape(_B).astype(jnp.int32)
    out = _sc_dot(uid, iid, user_table.T, item_table)
    return out.reshape(_B, 1)
